# Initial kernel scaffold; baseline (speedup 1.0000x reference)
import jax
import jax.numpy as jnp
from jax.experimental import pallas as pl


def kernel(x, A, W1, b1, W2, b2, W3, b3):
    raise NotImplementedError("write your pallas kernel here")


# speedup vs baseline: 8.7467x; 8.7467x over previous
import functools

import jax
import jax.numpy as jnp
from jax import lax
from jax.experimental import pallas as pl
from jax.experimental.pallas import tpu as pltpu
from jax.experimental.pallas import tpu_sc as plsc

N = 10000
NP = 10240
E = 320000
ER = 2560
EPAD = ER * 128
NC, NS = 2, 16
RPT_ES = ER // (NC * NS)
RPT_FS = ER // NS
TROWS = NP // NS
BM = 1024

@functools.cache
def _mesh():
  return plsc.VectorSubcoreMesh(
      core_axis_name="c", subcore_axis_name="s", num_cores=NC, num_subcores=NS)


@functools.cache
def _make_agg(r_per_tile, src_core_off, dst_core_off):
  G = 16

  @functools.partial(
      pl.kernel,
      out_type=jax.ShapeDtypeStruct((2 * NP, 128), jnp.float32),
      mesh=_mesh(),
      scratch_types=[
          pltpu.VMEM((G, 128), jnp.int32),
          pltpu.VMEM((G, 128), jnp.int32),
          pltpu.VMEM((128, 128), jnp.float32),
          pltpu.VMEM_SHARED((NP, 128), jnp.float32),
          pltpu.SemaphoreType.DMA,
      ],
  )
  def agg(table_hbm, srcr_hbm, dstr_hbm, zeros_hbm, out_hbm,
          src_v, dst_v, gbuf, acc, sem):
    cid = lax.axis_index("c")
    sid = lax.axis_index("s")
    pltpu.sync_copy(zeros_hbm, gbuf)
    for k in range(TROWS // 128):
      pltpu.sync_copy(gbuf, acc.at[pl.ds(sid * TROWS + k * 128, 128)])
    plsc.subcore_barrier()
    src_base = cid * src_core_off + sid * r_per_tile
    dst_base = cid * dst_core_off + sid * r_per_tile

    def outer(g, carry):
      pltpu.sync_copy(srcr_hbm.at[pl.ds(src_base + g * G, G)], src_v)
      pltpu.sync_copy(dstr_hbm.at[pl.ds(dst_base + g * G, G)], dst_v)

      def body(j, c2):
        pltpu.async_copy(table_hbm.at[src_v.at[j]], gbuf, sem).wait()
        pltpu.sync_copy(gbuf, acc.at[dst_v.at[j]], add=True)
        return c2

      lax.fori_loop(0, G, body, 0)
      return carry

    lax.fori_loop(0, r_per_tile // G, outer, 0)
    plsc.subcore_barrier()
    for k in range(TROWS // 128):
      r0 = sid * TROWS + k * 128
      pltpu.sync_copy(acc.at[pl.ds(r0, 128)], gbuf)
      pltpu.sync_copy(gbuf, out_hbm.at[pl.ds(cid * NP + r0, 128)])

  return agg


@functools.cache
def _make_deg():
  @functools.partial(
      pl.kernel,
      out_type=jax.ShapeDtypeStruct((2 * NP,), jnp.float32),
      mesh=_mesh(),
      scratch_types=[
          pltpu.VMEM((RPT_ES, 128), jnp.int32),
          pltpu.VMEM((128,), jnp.float32),
          pltpu.VMEM((TROWS,), jnp.float32),
          pltpu.VMEM_SHARED((NP,), jnp.float32),
      ],
  )
  def deg_kernel(dstr_hbm, ones_hbm, zrow_hbm, out_hbm, dst_v, ones_v, zbuf, acc):
    cid = lax.axis_index("c")
    sid = lax.axis_index("s")
    pltpu.sync_copy(zrow_hbm, zbuf)
    pltpu.sync_copy(zbuf, acc.at[pl.ds(sid * TROWS, TROWS)])
    pltpu.sync_copy(ones_hbm, ones_v)
    pltpu.sync_copy(
        dstr_hbm.at[pl.ds((cid * NS + sid) * RPT_ES, RPT_ES)], dst_v)
    plsc.subcore_barrier()

    def body(j, carry):
      pltpu.sync_copy(ones_v, acc.at[dst_v.at[j]], add=True)
      return carry

    lax.fori_loop(0, RPT_ES, body, 0)
    plsc.subcore_barrier()
    pltpu.sync_copy(acc.at[pl.ds(sid * TROWS, TROWS)], zbuf)
    pltpu.sync_copy(zbuf, out_hbm.at[pl.ds(cid * NP + sid * TROWS, TROWS)])

  return deg_kernel


def _k0_body(deg0_ref, deg1_ref, x_ref, dinv_ref, s0_ref):
  dinv = lax.rsqrt(deg0_ref[...] + deg1_ref[...] + 1.0)
  dinv_ref[...] = dinv
  s0_ref[...] = x_ref[...] * dinv


def _k1_body(dinv_ref, s0_ref, p_ref, w1_ref, b1_ref, w2_ref,
             out1_ref, h2_ref):
  dinv = dinv_ref[...]
  a = (s0_ref[...] + p_ref[0] + p_ref[1]) * dinv
  o1 = jnp.maximum(
      jnp.dot(a, w1_ref[...], preferred_element_type=jnp.float32)
      + b1_ref[...], 0.0)
  out1_ref[...] = o1
  h2 = jnp.dot(o1 * dinv, w2_ref[...], preferred_element_type=jnp.float32)
  h2_ref[0] = h2[:, :128]
  h2_ref[1] = h2[:, 128:]


def _k2_body(dinv_ref, agg_ref, h2_ref, b2_ref, w3_ref, out2_ref, h3_ref):
  dinv = dinv_ref[...]
  t0 = jnp.maximum((agg_ref[0] + h2_ref[0]) * dinv + b2_ref[0], 0.0)
  t1 = jnp.maximum((agg_ref[1] + h2_ref[1]) * dinv + b2_ref[1], 0.0)
  out2_ref[...] = jnp.concatenate([t0, t1], axis=1)
  h3_ref[...] = (
      jnp.dot(t0 * dinv, w3_ref[0], preferred_element_type=jnp.float32)
      + jnp.dot(t1 * dinv, w3_ref[1], preferred_element_type=jnp.float32))


def _k3_body(dinv_ref, h3_ref, q_ref, b3_ref, out3_ref):
  out3_ref[...] = jnp.maximum(
      (h3_ref[...] + q_ref[0] + q_ref[1]) * dinv_ref[...] + b3_ref[...], 0.0)


def _row_spec(width):
  return pl.BlockSpec((BM, width), lambda i: (i, 0))


def _half_spec():
  return pl.BlockSpec((2, BM, 128), lambda i: (0, i, 0))


def _full_spec(shape):
  return pl.BlockSpec(shape, lambda i: tuple(0 for _ in shape))


_GRID = (NP // BM,)
_f32 = jnp.float32

_k0_call = pl.pallas_call(
    _k0_body,
    grid=_GRID,
    in_specs=[_row_spec(1), _row_spec(1), _row_spec(128)],
    out_specs=[_row_spec(1), _row_spec(128)],
    out_shape=[jax.ShapeDtypeStruct((NP, 1), _f32),
               jax.ShapeDtypeStruct((NP, 128), _f32)],
)

_k1_call = pl.pallas_call(
    _k1_body,
    grid=_GRID,
    in_specs=[_row_spec(1), _row_spec(128), _half_spec(),
              _full_spec((128, 512)), _full_spec((1, 512)),
              _full_spec((512, 256))],
    out_specs=[_row_spec(512), _half_spec()],
    out_shape=[jax.ShapeDtypeStruct((NP, 512), _f32),
               jax.ShapeDtypeStruct((2, NP, 128), _f32)],
)

_k2_call = pl.pallas_call(
    _k2_body,
    grid=_GRID,
    in_specs=[_row_spec(1), _half_spec(), _half_spec(),
              _full_spec((2, 1, 128)), _full_spec((2, 128, 128))],
    out_specs=[_row_spec(256), _row_spec(128)],
    out_shape=[jax.ShapeDtypeStruct((NP, 256), _f32),
               jax.ShapeDtypeStruct((NP, 128), _f32)],
)

_k3_call = pl.pallas_call(
    _k3_body,
    grid=_GRID,
    in_specs=[_row_spec(1), _row_spec(128), _half_spec(),
              _full_spec((1, 128))],
    out_specs=_row_spec(128),
    out_shape=jax.ShapeDtypeStruct((NP, 128), _f32),
)


def kernel(x, A, W1, b1, W2, b2, W3, b3):
  xp = jnp.zeros((NP, 128), _f32).at[:N].set(x)
  pad_idx = jnp.full((EPAD - E,), N, jnp.int32)
  srcr = jnp.concatenate([A[0], pad_idx]).reshape(ER, 128)
  dstr = jnp.concatenate([A[1], pad_idx]).reshape(ER, 128)
  src_fs = jnp.concatenate([srcr, srcr + NP], axis=0)
  zeros_tile = jnp.zeros((128, 128), _f32)
  zeros_row = jnp.zeros((TROWS,), _f32)
  ones_row = jnp.ones((128,), _f32)

  deg2 = _make_deg()(dstr, ones_row, zeros_row)
  deg0 = deg2[:NP].reshape(NP, 1)
  deg1 = deg2[NP:].reshape(NP, 1)
  dinv, s0 = _k0_call(deg0, deg1, xp)

  agg_es = _make_agg(RPT_ES, NS * RPT_ES, NS * RPT_ES)
  agg_fs = _make_agg(RPT_FS, ER, 0)

  p1 = agg_es(s0, srcr, dstr, zeros_tile).reshape(2, NP, 128)
  out1, h2 = _k1_call(dinv, s0, p1, W1, b1.reshape(1, 512), W2)

  agg2 = agg_fs(h2.reshape(2 * NP, 128), src_fs, dstr,
                zeros_tile).reshape(2, NP, 128)
  out2, h3 = _k2_call(dinv, agg2, h2, b2.reshape(2, 1, 128),
                      W3.reshape(2, 128, 128))

  p3 = agg_es(h3, srcr, dstr, zeros_tile).reshape(2, NP, 128)
  out3 = _k3_call(dinv, h3, p3, b3.reshape(1, 128))

  return (out1[:N], out2[:N], out3[:N])
